# baseline (speedup 1.0000x reference)
import functools

import jax
import jax.numpy as jnp
import numpy as np
from jax import lax
from jax.experimental import pallas as pl
from jax.experimental.pallas import tpu as pltpu
from jax.experimental.pallas import tpu_sc as plsc

_B = 512
_NPQ = 1024
_NCB = 128
_NCW = 256
_SUB = 6
_EMB = _NCB * _SUB

_BB = 64

_NATIVE_COL = np.arange(_SUB).reshape(_SUB, 1) + _SUB * np.arange(_NCB)
_NATIVE_COL = _NATIVE_COL.reshape(_EMB)
_S200 = np.zeros((_EMB, _EMB), np.float32)
_S200[_NATIVE_COL, np.arange(_EMB)] = 200.0
_ST = np.zeros((_EMB, _EMB), np.float32)
_ST[np.arange(_EMB), _NATIVE_COL] = 1.0


def _pq_select_body(v_ref, p_ref, n1_ref, n2t_ref, simi_ref, idx_ref):
    v = v_ref[...]
    ip = lax.dot_general(
        v, p_ref[...], (((1,), (1,)), ((), ())),
        preferred_element_type=jnp.float32)
    simi = -(n1_ref[...] + n2t_ref[...] - 2.0 * ip)
    simi_ref[...] = simi
    m = jnp.max(simi, axis=-1, keepdims=True)
    iota = lax.broadcasted_iota(jnp.int32, simi.shape, 1)
    idx_ref[...] = jnp.min(
        jnp.where(simi == m, iota, _NPQ), axis=-1, keepdims=True)


def _pq_select(vectors, p_t, n1, n2t):
    return pl.pallas_call(
        _pq_select_body,
        out_shape=(
            jax.ShapeDtypeStruct((_B, _NPQ), jnp.float32),
            jax.ShapeDtypeStruct((_B, 1), jnp.int32),
        ),
    )(vectors, p_t, n1, n2t)


def _gather_rows_sc(table, idx):
    rows, d = table.shape
    (b,) = idx.shape
    info = plsc.get_sparse_core_info()
    nw = info.num_cores * info.num_subcores
    b_per_w = b // nw
    mesh = plsc.VectorSubcoreMesh(core_axis_name="c", subcore_axis_name="s")

    @functools.partial(
        pl.kernel,
        mesh=mesh,
        out_type=jax.ShapeDtypeStruct((b, d), jnp.float32),
        scratch_types=[
            pltpu.VMEM((b_per_w,), jnp.int32),
            pltpu.VMEM((b_per_w, d), jnp.float32),
            pltpu.SemaphoreType.DMA,
        ],
    )
    def k(table_hbm, idx_hbm, out_hbm, idx_v, rows_v, sem):
        wid = lax.axis_index("s") * info.num_cores + lax.axis_index("c")
        base = wid * b_per_w
        pltpu.sync_copy(idx_hbm.at[pl.ds(base, b_per_w)], idx_v)
        pltpu.async_copy(table_hbm.at[idx_v], rows_v, sem).wait()
        pltpu.sync_copy(rows_v, out_hbm.at[pl.ds(base, b_per_w)])

    return k(table, idx)


def _code_select_body(v_ref, c_ref, cb_ref, s2_ref, st_ref, ca_ref, q_ref):
    c = c_ref[...]
    d = v_ref[...] - c
    r2 = lax.dot_general(d, s2_ref[...], (((1,), (0,)), ((), ())),
                         preferred_element_type=jnp.float32,
                         precision=lax.Precision.HIGHEST)
    csm = lax.dot_general(c, s2_ref[...], (((1,), (0,)), ((), ())),
                          preferred_element_type=jnp.float32,
                          precision=lax.Precision.HIGHEST) * 0.005
    cn2 = None
    for s in range(_SUB):
        cw = cb_ref[s]
        cn2 = cw * cw if cn2 is None else cn2 + cw * cw
    x = cn2 * -100.0
    for s in range(_SUB):
        x = x + r2[:, s * _NCB:(s + 1) * _NCB][:, :, None] * cb_ref[s][None]
    m = jnp.max(x, axis=-1, keepdims=True)
    e = jnp.exp(x - m)
    rcp = 1.0 / jnp.sum(e, axis=-1, keepdims=True)
    ca_ref[...] = e * rcp
    iota = lax.broadcasted_iota(jnp.int32, x.shape, 2)
    idx = jnp.min(jnp.where(x == m, iota, _NCW), axis=-1)
    idx_t = idx.T
    hi = idx_t >= 128
    idx_lo = jnp.where(hi, 0, idx_t)
    idx_hi = jnp.where(hi, idx_t - 128, 0)
    qn_t = []
    for s in range(_SUB):
        g_lo = jnp.take_along_axis(cb_ref[s][:, :128], idx_lo, axis=1)
        g_hi = jnp.take_along_axis(cb_ref[s][:, 128:], idx_hi, axis=1)
        quant_t = jnp.where(hi, g_hi, g_lo)
        qn_t.append(csm[:, s * _NCB:(s + 1) * _NCB].T + quant_t)
    sumsq = qn_t[0] * qn_t[0]
    for s in range(1, _SUB):
        sumsq = sumsq + qn_t[s] * qn_t[s]
    denom = jnp.clip(jnp.sqrt(jnp.sum(sumsq, axis=0, keepdims=True)),
                     1e-12, None)
    qsm = jnp.concatenate([t.T for t in qn_t], axis=1) / denom.T
    q_ref[...] = lax.dot_general(qsm, st_ref[...], (((1,), (0,)), ((), ())),
                                 preferred_element_type=jnp.float32,
                                 precision=lax.Precision.HIGHEST)


def _code_select(vectors, pq_centers, cbt):
    nblk = _B // _BB
    return pl.pallas_call(
        _code_select_body,
        grid=(nblk,),
        in_specs=[
            pl.BlockSpec((_BB, _EMB), lambda i: (i, 0)),
            pl.BlockSpec((_BB, _EMB), lambda i: (i, 0)),
            pl.BlockSpec((_SUB, _NCB, _NCW), lambda i: (0, 0, 0)),
            pl.BlockSpec((_EMB, _EMB), lambda i: (0, 0)),
            pl.BlockSpec((_EMB, _EMB), lambda i: (0, 0)),
        ],
        out_specs=(
            pl.BlockSpec((_BB, _NCB, _NCW), lambda i: (i, 0, 0)),
            pl.BlockSpec((_BB, _EMB), lambda i: (i, 0)),
        ),
        out_shape=(
            jax.ShapeDtypeStruct((_B, _NCB, _NCW), jnp.float32),
            jax.ShapeDtypeStruct((_B, _EMB), jnp.float32),
        ),
    )(vectors, pq_centers, cbt, jnp.asarray(_S200), jnp.asarray(_ST))


def kernel(vectors, pq_assgin_layer, codebooks):
    n1 = jnp.sum(vectors ** 2, axis=-1, keepdims=True)
    n2 = jnp.sum(pq_assgin_layer ** 2, axis=-1, keepdims=True)
    simi, idx = _pq_select(vectors, pq_assgin_layer, n1, n2.T)
    pq_centers = _gather_rows_sc(pq_assgin_layer, idx.reshape(_B))
    cbt = codebooks.transpose(2, 0, 1)
    ca, q = _code_select(vectors, pq_centers, cbt)
    return (simi, pq_centers, ca, q)


# speedup vs baseline: 1.1560x; 1.1560x over previous
import functools

import jax
import jax.numpy as jnp
import numpy as np
from jax import lax
from jax.experimental import pallas as pl
from jax.experimental.pallas import tpu as pltpu
from jax.experimental.pallas import tpu_sc as plsc

_B = 512
_NPQ = 1024
_NCB = 128
_NCW = 256
_SUB = 6
_EMB = _NCB * _SUB

_BB = 64

def _pq_select_body(v_ref, p_ref, n1_ref, n2t_ref, simi_ref, idx_ref):
    v = v_ref[...]
    ip = lax.dot_general(
        v, p_ref[...], (((1,), (1,)), ((), ())),
        preferred_element_type=jnp.float32)
    simi = -(n1_ref[...] + n2t_ref[...] - 2.0 * ip)
    simi_ref[...] = simi
    m = jnp.max(simi, axis=-1, keepdims=True)
    iota = lax.broadcasted_iota(jnp.int32, simi.shape, 1)
    idx_ref[...] = jnp.min(
        jnp.where(simi == m, iota, _NPQ), axis=-1, keepdims=True)


def _pq_select(vectors, p_t, n1, n2t):
    return pl.pallas_call(
        _pq_select_body,
        out_shape=(
            jax.ShapeDtypeStruct((_B, _NPQ), jnp.float32),
            jax.ShapeDtypeStruct((_B, 1), jnp.int32),
        ),
    )(vectors, p_t, n1, n2t)


def _gather_rows_sc(table, idx):
    rows, d = table.shape
    (b,) = idx.shape
    info = plsc.get_sparse_core_info()
    nw = info.num_cores * info.num_subcores
    b_per_w = b // nw
    mesh = plsc.VectorSubcoreMesh(core_axis_name="c", subcore_axis_name="s")

    @functools.partial(
        pl.kernel,
        mesh=mesh,
        out_type=jax.ShapeDtypeStruct((b, d), jnp.float32),
        scratch_types=[
            pltpu.VMEM((b_per_w,), jnp.int32),
            pltpu.VMEM((b_per_w, d), jnp.float32),
            pltpu.SemaphoreType.DMA,
        ],
    )
    def k(table_hbm, idx_hbm, out_hbm, idx_v, rows_v, sem):
        wid = lax.axis_index("s") * info.num_cores + lax.axis_index("c")
        base = wid * b_per_w
        pltpu.sync_copy(idx_hbm.at[pl.ds(base, b_per_w)], idx_v)
        pltpu.async_copy(table_hbm.at[idx_v], rows_v, sem).wait()
        pltpu.sync_copy(rows_v, out_hbm.at[pl.ds(base, b_per_w)])

    return k(table, idx)


def _code_select_body(vs_ref, cs_ref, cb_ref, ca_ref, qs_ref):
    r2 = [(vs_ref[s] - cs_ref[s]) * 200.0 for s in range(_SUB)]
    cn2 = None
    for s in range(_SUB):
        c = cb_ref[s]
        cn2 = c * c if cn2 is None else cn2 + c * c
    x = cn2 * -100.0
    for s in range(_SUB):
        x = x + r2[s][:, :, None] * cb_ref[s][None, :, :]
    m = jnp.max(x, axis=-1, keepdims=True)
    e = jnp.exp(x - m)
    rcp = 1.0 / jnp.sum(e, axis=-1, keepdims=True)
    ca_ref[...] = e * rcp
    iota = lax.broadcasted_iota(jnp.int32, x.shape, 2).astype(jnp.float32)
    idx_f = jnp.min(jnp.where(x == m, iota, float(_NCW)), axis=-1)
    idx_t = idx_f.T.astype(jnp.int32)
    hi = idx_t >= 128
    idx_lo = jnp.where(hi, 0, idx_t)
    idx_hi = jnp.where(hi, idx_t - 128, 0)
    qn_t = []
    for s in range(_SUB):
        g_lo = jnp.take_along_axis(cb_ref[s][:, :128], idx_lo, axis=1)
        g_hi = jnp.take_along_axis(cb_ref[s][:, 128:], idx_hi, axis=1)
        quant_t = jnp.where(hi, g_hi, g_lo)
        qn_t.append(cs_ref[s].T + quant_t)
    sumsq = qn_t[0] * qn_t[0]
    for s in range(1, _SUB):
        sumsq = sumsq + qn_t[s] * qn_t[s]
    denom = jnp.clip(jnp.sqrt(jnp.sum(sumsq, axis=0, keepdims=True)),
                     1e-12, None)
    for s in range(_SUB):
        qs_ref[0, s] = qn_t[s] / denom


def _code_select(vs, cs, cbt):
    nblk = _B // _BB
    return pl.pallas_call(
        _code_select_body,
        grid=(nblk,),
        in_specs=[
            pl.BlockSpec((_SUB, _BB, _NCB), lambda i: (0, i, 0)),
            pl.BlockSpec((_SUB, _BB, _NCB), lambda i: (0, i, 0)),
            pl.BlockSpec((_SUB, _NCB, _NCW), lambda i: (0, 0, 0)),
        ],
        out_specs=(
            pl.BlockSpec((_BB, _NCB, _NCW), lambda i: (i, 0, 0)),
            pl.BlockSpec((1, _SUB, _NCB, _BB), lambda i: (i, 0, 0, 0)),
        ),
        out_shape=(
            jax.ShapeDtypeStruct((_B, _NCB, _NCW), jnp.float32),
            jax.ShapeDtypeStruct((_B // _BB, _SUB, _NCB, _BB), jnp.float32),
        ),
    )(vs, cs, cbt)


def kernel(vectors, pq_assgin_layer, codebooks):
    n1 = jnp.sum(vectors ** 2, axis=-1, keepdims=True)
    n2 = jnp.sum(pq_assgin_layer ** 2, axis=-1, keepdims=True)
    simi, idx = _pq_select(vectors, pq_assgin_layer, n1, n2.T)
    pq_centers = _gather_rows_sc(pq_assgin_layer, idx.reshape(_B))
    vs = vectors.reshape(_B, _NCB, _SUB).transpose(2, 0, 1)
    cs = pq_centers.reshape(_B, _NCB, _SUB).transpose(2, 0, 1)
    cbt = codebooks.transpose(2, 0, 1)
    ca, qs = _code_select(vs, cs, cbt)
    q = qs.transpose(0, 3, 2, 1).reshape(_B, _EMB)
    return (simi, pq_centers, ca, q)
